# baseline (speedup 1.0000x reference)
import functools

import jax
import jax.numpy as jnp
from jax import lax
from jax.experimental import pallas as pl
from jax.experimental.pallas import tpu as pltpu
from jax.experimental.pallas import tpu_sc as plsc

_NC = 2
_NS = 16
_NW = _NC * _NS


def _sc_broadcast(table, n_batch, length):
    V, D = table.shape
    rows_per_w = length // _NW
    CHUNK = 64
    n_chunks = rows_per_w // CHUNK

    mesh = plsc.VectorSubcoreMesh(core_axis_name="c", subcore_axis_name="s")

    @functools.partial(
        pl.kernel,
        out_type=jax.ShapeDtypeStruct((n_batch, length, D), table.dtype),
        mesh=mesh,
        scratch_types=[
            pltpu.VMEM((CHUNK, D), table.dtype),
            pltpu.SemaphoreType.DMA,
        ],
    )
    def sc_copy(table_hbm, out_hbm, buf, ssem):
        wid = lax.axis_index("s") * _NC + lax.axis_index("c")
        base = wid * rows_per_w
        for c in range(n_chunks):
            off = base + c * CHUNK
            pltpu.sync_copy(table_hbm.at[pl.ds(off, CHUNK)], buf)
            stores = [
                pltpu.async_copy(
                    buf, out_hbm.at[b, pl.ds(off, CHUNK)], ssem)
                for b in range(n_batch)
            ]
            for h in stores:
                h.wait()

    return sc_copy(table)


def _tc_broadcast(table, n_batch, length):
    V, D = table.shape
    BLK = 512
    return pl.pallas_call(
        _copy_body,
        grid=(length // BLK, n_batch),
        in_specs=[pl.BlockSpec((BLK, D), lambda i, b: (i, 0))],
        out_specs=pl.BlockSpec((1, BLK, D), lambda i, b: (b, i, 0)),
        out_shape=jax.ShapeDtypeStruct((n_batch, length, D), table.dtype),
    )(table)


def kernel(x, table):
    B, length, _ = x.shape
    half = B // 2
    out_sc = _sc_broadcast(table, half, length)
    out_tc = _tc_broadcast(table, B - half, length)
    return jnp.concatenate([out_sc, out_tc], axis=0)


def _kernel_sc_only(x, table):
    B, length, _ = x.shape
    V, D = table.shape
    rows_per_w = length // _NW
    CHUNK = 64
    n_chunks = rows_per_w // CHUNK

    mesh = plsc.VectorSubcoreMesh(core_axis_name="c", subcore_axis_name="s")

    @functools.partial(
        pl.kernel,
        out_type=jax.ShapeDtypeStruct((B, length, D), table.dtype),
        mesh=mesh,
        scratch_types=[
            pltpu.VMEM((CHUNK, D), table.dtype),
            pltpu.SemaphoreType.DMA,
        ],
    )
    def sc_copy(table_hbm, out_hbm, buf, ssem):
        wid = lax.axis_index("s") * _NC + lax.axis_index("c")
        base = wid * rows_per_w
        for c in range(n_chunks):
            off = base + c * CHUNK
            pltpu.sync_copy(table_hbm.at[pl.ds(off, CHUNK)], buf)
            stores = [
                pltpu.async_copy(
                    buf, out_hbm.at[b, pl.ds(off, CHUNK)], ssem)
                for b in range(B)
            ]
            for h in stores:
                h.wait()

    return sc_copy(table)


def _copy_body(t_ref, o_ref):
    o_ref[0] = t_ref[...]


def _kernel_tc(x, table):
    B, length, _ = x.shape
    _, D = table.shape
    BLK = 512
    out = pl.pallas_call(
        _copy_body,
        grid=(length // BLK, B),
        in_specs=[pl.BlockSpec((BLK, D), lambda i, b: (i, 0))],
        out_specs=pl.BlockSpec((1, BLK, D), lambda i, b: (b, i, 0)),
        out_shape=jax.ShapeDtypeStruct((B, length, D), table.dtype),
    )(table)
    return out


# speedup vs baseline: 2.2353x; 2.2353x over previous
import functools

import jax
import jax.numpy as jnp
from jax import lax
from jax.experimental import pallas as pl
from jax.experimental.pallas import tpu as pltpu
from jax.experimental.pallas import tpu_sc as plsc

_NC = 2
_NS = 16
_NW = _NC * _NS


def kernel(x, table):
    B, length, _ = x.shape
    V, D = table.shape
    rows_per_w = length // _NW
    CHUNK = 32
    NBUF = 3
    n_chunks = rows_per_w // CHUNK

    mesh = plsc.VectorSubcoreMesh(core_axis_name="c", subcore_axis_name="s")

    @functools.partial(
        pl.kernel,
        out_type=jax.ShapeDtypeStruct((B, length, D), table.dtype),
        mesh=mesh,
        scratch_types=(
            [pltpu.VMEM((CHUNK, D), table.dtype) for _ in range(NBUF)]
            + [pltpu.SemaphoreType.DMA]
            + [pltpu.SemaphoreType.DMA for _ in range(NBUF)]
        ),
    )
    def sc_copy(table_hbm, out_hbm, *refs):
        bufs = refs[:NBUF]
        lsem = refs[NBUF]
        ssems = refs[NBUF + 1:]
        wid = lax.axis_index("s") * _NC + lax.axis_index("c")
        base = wid * rows_per_w

        def start_load(c):
            off = base + c * CHUNK
            return pltpu.async_copy(
                table_hbm.at[pl.ds(off, CHUNK)], bufs[c % NBUF], lsem)

        loads = {}
        stores = {}
        for c in range(min(NBUF, n_chunks)):
            loads[c] = start_load(c)
        for c in range(n_chunks):
            k = c % NBUF
            loads[c].wait()
            off = base + c * CHUNK
            stores[c] = [
                pltpu.async_copy(
                    bufs[k], out_hbm.at[b, pl.ds(off, CHUNK)], ssems[k])
                for b in range(B)
            ]
            nxt = c + 1
            if NBUF <= nxt < n_chunks:
                for h in stores[nxt - NBUF]:
                    h.wait()
                loads[nxt] = start_load(nxt)
        for c in range(max(0, n_chunks - NBUF), n_chunks):
            for h in stores[c]:
                h.wait()

    return sc_copy(table)
